# Initial kernel scaffold; baseline (speedup 1.0000x reference)
import jax
import jax.numpy as jnp
from jax.experimental import pallas as pl


def kernel(embed_table, image_embeds, before_ids, after_ids, bos_id):
    raise NotImplementedError("write your pallas kernel here")


# speedup vs baseline: 1.2937x; 1.2937x over previous
import functools

import jax
import jax.numpy as jnp
import numpy as np
from jax import lax
from jax.experimental import pallas as pl
from jax.experimental.pallas import tpu as pltpu
from jax.experimental.pallas import tpu_sc as plsc

D = 2048
SEQ_IMG = 729
N_TOK = 4097
SEQ_OUT = N_TOK + SEQ_IMG

NW = 32
SUB = 24
NCHUNK = 6
CHUNK = SUB * NCHUNK
TOK_PAD = NW * CHUNK
IMG_PAD = NW * SUB

_TOK_DST = np.zeros((TOK_PAD,), np.int32)
_TOK_DST[:2049] = np.arange(2049)
_TOK_DST[2049:N_TOK] = np.arange(2778, 4826)
_IMG_SRC = np.zeros((IMG_PAD,), np.int32)
_IMG_SRC[:SEQ_IMG] = np.arange(SEQ_IMG)
_IMG_DST = np.full((IMG_PAD,), 2049, np.int32)
_IMG_DST[:SEQ_IMG] = np.arange(2049, 2778)


def _sc_body(table_hbm, img_hbm, tok_src_hbm, tok_dst_hbm, img_src_hbm,
             img_dst_hbm, out_hbm, sidx_v, didx_v, iidx_v, idst_v, buf_v,
             sem_in, sem_out):
    c = lax.axis_index("c")
    s = lax.axis_index("s")
    w = c * 16 + s
    base = w * CHUNK

    for j in range(NCHUNK):
        pltpu.sync_copy(tok_src_hbm.at[pl.ds(base + j * SUB, SUB)], sidx_v.at[j])
        pltpu.sync_copy(tok_dst_hbm.at[pl.ds(base + j * SUB, SUB)], didx_v.at[j])
    pltpu.sync_copy(img_src_hbm.at[pl.ds(w * SUB, SUB)], iidx_v)
    pltpu.sync_copy(img_dst_hbm.at[pl.ds(w * SUB, SUB)], idst_v)

    for j in range(NCHUNK):
        pltpu.async_copy(table_hbm.at[sidx_v.at[j]], buf_v, sem_in).wait()
        pltpu.async_copy(buf_v, out_hbm.at[didx_v.at[j]], sem_out).wait()

    pltpu.async_copy(img_hbm.at[iidx_v], buf_v, sem_in).wait()
    pltpu.async_copy(buf_v, out_hbm.at[idst_v], sem_out).wait()


@functools.partial(
    pl.kernel,
    mesh=plsc.VectorSubcoreMesh(core_axis_name="c", subcore_axis_name="s"),
    out_type=jax.ShapeDtypeStruct((SEQ_OUT, D), jnp.float32),
    scratch_types=[
        pltpu.VMEM((NCHUNK, SUB), jnp.int32),
        pltpu.VMEM((NCHUNK, SUB), jnp.int32),
        pltpu.VMEM((SUB,), jnp.int32),
        pltpu.VMEM((SUB,), jnp.int32),
        pltpu.VMEM((SUB, D), jnp.float32),
        pltpu.SemaphoreType.DMA,
        pltpu.SemaphoreType.DMA,
    ],
)
def _sc_gather(*refs):
    _sc_body(*refs)


def kernel(embed_table, image_embeds, before_ids, after_ids, bos_id):
    bos = jnp.asarray(bos_id, jnp.int32)
    tok_src = jnp.concatenate([
        bos[None],
        before_ids[0].astype(jnp.int32),
        after_ids[0].astype(jnp.int32),
        jnp.full((TOK_PAD - N_TOK,), bos, jnp.int32),
    ])
    out = _sc_gather(
        embed_table,
        image_embeds[0],
        tok_src,
        jnp.asarray(_TOK_DST),
        jnp.asarray(_IMG_SRC),
        jnp.asarray(_IMG_DST),
    )
    return out[None]


# speedup vs baseline: 2.2029x; 1.7028x over previous
import functools

import jax
import jax.numpy as jnp
import numpy as np
from jax import lax
from jax.experimental import pallas as pl
from jax.experimental.pallas import tpu as pltpu
from jax.experimental.pallas import tpu_sc as plsc

D = 2048
SEQ_IMG = 729
N_TOK = 4097
SEQ_OUT = N_TOK + SEQ_IMG

NW = 32
SUB = 24
NCHUNK = 6
CHUNK = SUB * NCHUNK
TOK_PAD = NW * CHUNK
IMG_PAD = NW * SUB

_TOK_DST = np.empty((TOK_PAD,), np.int32)
_TOK_DST[:2049] = np.arange(2049)
_TOK_DST[2049:N_TOK] = np.arange(2778, 4826)
_TOK_DST[N_TOK:] = _TOK_DST[1:1 + TOK_PAD - N_TOK]
_IMG_SRC = np.empty((IMG_PAD,), np.int32)
_IMG_SRC[:SEQ_IMG] = np.arange(SEQ_IMG)
_IMG_SRC[SEQ_IMG:] = _IMG_SRC[:IMG_PAD - SEQ_IMG]
_IMG_DST = np.empty((IMG_PAD,), np.int32)
_IMG_DST[:SEQ_IMG] = np.arange(2049, 2778)
_IMG_DST[SEQ_IMG:] = _IMG_DST[:IMG_PAD - SEQ_IMG]


def _sc_body(table_hbm, img_hbm, tok_src_hbm, tok_dst_hbm, img_src_hbm,
             img_dst_hbm, out_hbm, sidx_v, didx_v, iidx_v, idst_v, buf0, buf1,
             sem_in0, sem_in1, sem_out0, sem_out1):
    c = lax.axis_index("c")
    s = lax.axis_index("s")
    w = c * 16 + s

    pltpu.sync_copy(tok_src_hbm.at[w], sidx_v)
    pltpu.sync_copy(tok_dst_hbm.at[w], didx_v)
    pltpu.sync_copy(img_src_hbm.at[w], iidx_v)
    pltpu.sync_copy(img_dst_hbm.at[w], idst_v)

    chunks = [(table_hbm, sidx_v.at[j], didx_v.at[j]) for j in range(NCHUNK)]
    chunks.append((img_hbm, iidx_v.at[0], idst_v.at[0]))
    n = len(chunks)
    bufs = (buf0, buf1)
    sin = (sem_in0, sem_in1)
    sout = (sem_out0, sem_out1)

    gat = [None] * n
    sca = [None] * n
    src0, sidx0, _ = chunks[0]
    gat[0] = pltpu.async_copy(src0.at[sidx0], bufs[0], sin[0])
    for j in range(n):
        p = j % 2
        gat[j].wait()
        sca[j] = pltpu.async_copy(bufs[p], out_hbm.at[chunks[j][2]], sout[p])
        if j + 1 < n:
            if j >= 1:
                sca[j - 1].wait()
            src, sidx, _ = chunks[j + 1]
            gat[j + 1] = pltpu.async_copy(src.at[sidx], bufs[(j + 1) % 2],
                                          sin[(j + 1) % 2])
    sca[n - 2].wait()
    sca[n - 1].wait()


@functools.partial(
    pl.kernel,
    mesh=plsc.VectorSubcoreMesh(core_axis_name="c", subcore_axis_name="s"),
    out_type=jax.ShapeDtypeStruct((SEQ_OUT, D), jnp.float32),
    scratch_types=[
        pltpu.VMEM((NCHUNK, SUB), jnp.int32),
        pltpu.VMEM((NCHUNK, SUB), jnp.int32),
        pltpu.VMEM((1, SUB), jnp.int32),
        pltpu.VMEM((1, SUB), jnp.int32),
        pltpu.VMEM((SUB, D), jnp.float32),
        pltpu.VMEM((SUB, D), jnp.float32),
        pltpu.SemaphoreType.DMA,
        pltpu.SemaphoreType.DMA,
        pltpu.SemaphoreType.DMA,
        pltpu.SemaphoreType.DMA,
    ],
)
def _sc_gather(*refs):
    _sc_body(*refs)


def kernel(embed_table, image_embeds, before_ids, after_ids, bos_id):
    bos = jnp.asarray(bos_id, jnp.int32)
    tok_src = jnp.concatenate([
        bos[None],
        before_ids[0].astype(jnp.int32),
        after_ids[0].astype(jnp.int32),
    ])
    tok_src = jnp.concatenate([tok_src, tok_src[1:1 + TOK_PAD - N_TOK]])
    out = _sc_gather(
        embed_table,
        image_embeds[0],
        tok_src.reshape(NW, NCHUNK, SUB),
        jnp.asarray(_TOK_DST.reshape(NW, NCHUNK, SUB)),
        jnp.asarray(_IMG_SRC.reshape(NW, 1, SUB)),
        jnp.asarray(_IMG_DST.reshape(NW, 1, SUB)),
    )
    return out[None]
